# Initial kernel scaffold; baseline (speedup 1.0000x reference)
import jax
import jax.numpy as jnp
from jax.experimental import pallas as pl


def kernel(x, edge_index, edge_attr, batch, in_W, in_b, ep_W, W1, b1, W2, b2, ln_g, ln_b, bn_g, bn_b, ro_W, ro_b):
    raise NotImplementedError("write your pallas kernel here")


# speedup vs baseline: 3.6365x; 3.6365x over previous
import functools

import numpy as np
import jax
import jax.numpy as jnp
from jax import lax
from jax.experimental import pallas as pl
from jax.experimental.pallas import tpu as pltpu
from jax.experimental.pallas import tpu_sc as plsc

N = 10000
E = 320000
NODE_IN = 128
EDGE_IN = 16
H = 256
HH = 128
G = 64
DEPTH = 5

NC, NS, L = 2, 16, 16
K = 40
NBUF = 5
NSB = 2
PH = 50
IU = 10
EPS = E // NS
NPH = EPS // (K * PH)
ZR = 200


def _in_proj_body(x_ref, w_ref, b_ref, o0, o1):
    z = jnp.dot(x_ref[...], w_ref[...], preferred_element_type=jnp.float32)
    z = z + b_ref[...]
    o0[...] = z[:, :HH]
    o1[...] = z[:, HH:]


def _in_proj(x, in_W, in_b):
    bn = 2000
    return pl.pallas_call(
        _in_proj_body,
        grid=(N // bn,),
        in_specs=[
            pl.BlockSpec((bn, NODE_IN), lambda i: (i, 0)),
            pl.BlockSpec((NODE_IN, H), lambda i: (0, 0)),
            pl.BlockSpec((1, H), lambda i: (0, 0)),
        ],
        out_specs=[pl.BlockSpec((bn, HH), lambda i: (i, 0))] * 2,
        out_shape=[jax.ShapeDtypeStruct((N, HH), jnp.float32)] * 2,
    )(x, in_W, in_b.reshape(1, H))


def _pack_bf16(y):
    b = lax.bitcast_convert_type(
        y.astype(jnp.bfloat16).astype(jnp.float32), jnp.int32)
    lo = lax.shift_right_logical(b[:, :HH // 2], 16)
    hi = jnp.bitwise_and(b[:, HH // 2:], jnp.int32(-65536))
    return jnp.bitwise_or(lo, hi)


def _edge_proj_body(a_ref, w_ref, o0, o1):
    z = jnp.dot(a_ref[...], w_ref[...], preferred_element_type=jnp.float32)
    be = z.shape[0]
    o0[...] = _pack_bf16(z[:, :HH]).reshape(be // K, K, HH // 2)
    o1[...] = _pack_bf16(z[:, HH:]).reshape(be // K, K, HH // 2)


def _edge_proj(edge_attr, ep_W):
    be = 4000
    return pl.pallas_call(
        _edge_proj_body,
        grid=(E // be,),
        in_specs=[
            pl.BlockSpec((be, EDGE_IN), lambda i: (i, 0)),
            pl.BlockSpec((EDGE_IN, H), lambda i: (0, 0)),
        ],
        out_specs=[pl.BlockSpec((be // K, K, HH // 2),
                                lambda i: (i, 0, 0))] * 2,
        out_shape=[jax.ShapeDtypeStruct((E // K, K, HH // 2), jnp.int32)] * 2,
    )(edge_attr, ep_W)


def _mlp_body(h0, h1, a0, a1, w1, b1, w2, b2, lng, lnb, bng, bnb,
              o0, o1):
    h = jnp.concatenate([h0[...], h1[...]], axis=1)
    z = h + jnp.concatenate([a0[...], a1[...]], axis=1)
    z = jnp.dot(z, w1[...], preferred_element_type=jnp.float32)
    z = jnp.maximum(z + b1[...], 0.0)
    z = jnp.dot(z, w2[...], preferred_element_type=jnp.float32) + b2[...]
    m = jnp.mean(z, axis=1, keepdims=True)
    v = jnp.mean((z - m) ** 2, axis=1, keepdims=True)
    z = (z - m) * lax.rsqrt(v + 1e-5) * lng[...] + lnb[...]
    z = z * bng[...] + bnb[...]
    hn = h + jnp.maximum(z, 0.0)
    o0[...] = hn[:, :HH]
    o1[...] = hn[:, HH:]


def _mlp(h0, h1, a0, a1, w1, b1, w2, b2, lng, lnb, bng, bnb):
    bn = 2000
    vec = lambda: pl.BlockSpec((1, H), lambda i: (0, 0))
    half = lambda: pl.BlockSpec((bn, HH), lambda i: (i, 0))
    return pl.pallas_call(
        _mlp_body,
        grid=(N // bn,),
        in_specs=[half(), half(), half(), half(),
                  pl.BlockSpec((H, H), lambda i: (0, 0)), vec(),
                  pl.BlockSpec((H, H), lambda i: (0, 0)), vec(),
                  vec(), vec(), vec(), vec()],
        out_specs=[half()] * 2,
        out_shape=[jax.ShapeDtypeStruct((N, HH), jnp.float32)] * 2,
    )(h0, h1, a0, a1, w1, b1.reshape(1, H), w2, b2.reshape(1, H),
      lng.reshape(1, H), lnb.reshape(1, H), bng.reshape(1, H),
      bnb.reshape(1, H))


def _pool_body(b_ref, h0, h1, row, rob, o_ref, sums, cnts):
    i = pl.program_id(0)

    @pl.when(i == 0)
    def _():
        sums[...] = jnp.zeros_like(sums)
        cnts[...] = jnp.zeros_like(cnts)

    bb = b_ref[0, 0, :]
    bn = bb.shape[0]
    oh = (bb[None, :] == lax.broadcasted_iota(jnp.int32, (G, bn), 0))
    oh = oh.astype(jnp.float32)
    h = jnp.concatenate([h0[...], h1[...]], axis=1)
    sums[...] += jnp.dot(oh, h, preferred_element_type=jnp.float32)
    cnts[...] += jnp.sum(oh, axis=1, keepdims=True)

    @pl.when(i == pl.num_programs(0) - 1)
    def _():
        g = sums[...] / jnp.maximum(cnts[...][:, 0:1], 1.0)
        o_ref[...] = (jnp.dot(g, row[...], preferred_element_type=jnp.float32)
                      + rob[...])


def _pool_readout(batch, h0, h1, ro_W, ro_b):
    bn = 1000
    nb = N // bn
    half = lambda: pl.BlockSpec((bn, HH), lambda i: (i, 0))
    return pl.pallas_call(
        _pool_body,
        grid=(nb,),
        in_specs=[
            pl.BlockSpec((1, 1, bn), lambda i: (i, 0, 0)),
            half(), half(),
            pl.BlockSpec((H, H), lambda i: (0, 0)),
            pl.BlockSpec((1, H), lambda i: (0, 0)),
        ],
        out_specs=pl.BlockSpec((G, H), lambda i: (0, 0)),
        out_shape=jax.ShapeDtypeStruct((G, H), jnp.float32),
        scratch_shapes=[
            pltpu.VMEM((G, H), jnp.float32),
            pltpu.VMEM((G, 128), jnp.float32),
        ],
    )(batch.reshape(nb, 1, bn), h0, h1, ro_W, ro_b.reshape(1, H))


def _edge_sc_body(h0, h1, ea0, ea1, src5, dst5, o0, o1,
                  sidx, didx, m0, m1, m2, m3, m4, e0, e1,
                  sg0, sg1, sg2, sg3, sg4,
                  ss0, ss1, ss2, ss3, ss4, acc):
    c = lax.axis_index("c")
    s = lax.axis_index("s")
    bufB = (m0, m1, m2, m3, m4)
    bufE = (e0, e1)
    semG = (sg0, sg1, sg2, sg3, sg4)
    semS = (ss0, ss1, ss2, ss3, ss4)

    def run(h_hbm, ea_hbm, out_hbm):
        def zrow(r, _):
            for cc in range(HH // L):
                m0[r, pl.ds(cc * L, L)] = jnp.zeros((L,), jnp.float32)
            return 0

        lax.fori_loop(0, K, zrow, 0, unroll=2)
        for t in range((N // K + NS - 1) // NS):
            ch = s + t * NS

            @pl.when(ch < N // K)
            def _():
                pltpu.sync_copy(m0, acc.at[pl.ds(ch * K, K)])

        plsc.subcore_barrier()

        def in_start(g, jj, b, be):
            pltpu.async_copy(ea_hbm.at[s * (EPS // K) + g], bufE[be],
                             semG[b])
            pltpu.async_copy(h_hbm.at[sidx.at[jj]], bufB[b], semG[b])

        def in_wait(b, be):
            pltpu.make_async_copy(ea_hbm.at[0], bufE[be], semG[b]).wait()
            pltpu.make_async_copy(h_hbm.at[pl.ds(0, K)], bufB[b],
                                  semG[b]).wait()

        def s_start(jj, b):
            pltpu.async_copy(bufB[b], acc.at[didx.at[jj]], semS[b],
                             add=True)

        def s_wait(b):
            pltpu.make_async_copy(bufB[b], acc.at[pl.ds(0, K)],
                                  semS[b]).wait()

        M_HI = jnp.int32(-65536)

        def conv(b, be):
            sB, sE = bufB[b], bufE[be]

            def row(r, _):
                for cc in range(HH // 32):
                    iv = sE[r, pl.ds(cc * L, L)]
                    fe = plsc.bitcast(lax.shift_left(iv, 16), jnp.float32)
                    fo = plsc.bitcast(jnp.bitwise_and(iv, M_HI),
                                      jnp.float32)
                    lo = (r, pl.ds(cc * L, L))
                    hi = (r, pl.ds(64 + cc * L, L))
                    sB[lo] = jnp.maximum(sB[lo] + fe, 0.0)
                    sB[hi] = jnp.maximum(sB[hi] + fo, 0.0)
                return 0

            lax.fori_loop(0, K, row, 0, unroll=2)

        def phase(p, _):
            jb = p * PH
            pltpu.sync_copy(src5.at[s, p], sidx)
            pltpu.sync_copy(dst5.at[s, p], didx)
            in_start(jb + 0, 0, 0, 0)
            in_start(jb + 1, 1, 1, 1)

            def iter_t(t, _):
                for b in range(IU):
                    jj = t * IU + b
                    b5 = b % 5
                    bp5 = (b + 2) % 5
                    b2 = b % 2
                    in_wait(b5, b2)
                    conv(b5, b2)
                    s_start(jj, b5)
                    def do_c():
                        s_wait(bp5)

                    if b >= 3:
                        do_c()
                    else:
                        @pl.when(t > 0)
                        def _():
                            do_c()

                    if b < IU - 2:
                        in_start(jb + jj + 2, jj + 2, bp5, b2)
                    else:
                        @pl.when(t < PH // IU - 1)
                        def _():
                            in_start(jb + jj + 2, jj + 2, bp5, b2)
                return 0

            lax.fori_loop(0, PH // IU, iter_t, 0)
            for m in (PH - 3, PH - 2, PH - 1):
                s_wait(m % 5)
            return 0

        lax.fori_loop(0, NPH, phase, 0)
        plsc.subcore_barrier()
        for t in range((N // ZR + NS - 1) // NS):
            ch = s + t * NS

            @pl.when(ch < N // ZR)
            def _():
                pltpu.sync_copy(acc.at[pl.ds(ch * ZR, ZR)],
                                out_hbm.at[pl.ds(ch * ZR, ZR)])

    @pl.when(c == 0)
    def _():
        run(h0, ea0, o0)

    @pl.when(c == 1)
    def _():
        run(h1, ea1, o1)


@functools.partial(
    pl.kernel,
    out_type=(jax.ShapeDtypeStruct((N, HH), jnp.float32),
              jax.ShapeDtypeStruct((N, HH), jnp.float32)),
    mesh=plsc.VectorSubcoreMesh(core_axis_name="c", subcore_axis_name="s"),
    compiler_params=pltpu.CompilerParams(needs_layout_passes=False),
    scratch_types=(
        [pltpu.VMEM((PH, K), jnp.int32)] * 2
        + [pltpu.VMEM((K, HH), jnp.float32)] * NBUF
        + [pltpu.VMEM((K, HH // 2), jnp.int32)] * 2
        + [pltpu.SemaphoreType.DMA] * (2 * NBUF)
        + [pltpu.VMEM_SHARED((N, HH), jnp.float32)]
    ),
)
def _edge_phase(h0, h1, ea0, ea1, src5, dst5, o0, o1,
                sidx, didx, m0, m1, m2, m3, m4, e0, e1,
                sg0, sg1, sg2, sg3, sg4,
                ss0, ss1, ss2, ss3, ss4, acc):
    _edge_sc_body(h0, h1, ea0, ea1, src5, dst5, o0, o1,
                  sidx, didx, m0, m1, m2, m3, m4, e0, e1,
                  sg0, sg1, sg2, sg3, sg4,
                  ss0, ss1, ss2, ss3, ss4, acc)


@jax.jit
def kernel(x, edge_index, edge_attr, batch, in_W, in_b, ep_W, W1, b1,
           W2, b2, ln_g, ln_b, bn_g, bn_b, ro_W, ro_b):
    src5 = edge_index[0].astype(jnp.int32).reshape(NS, NPH, PH, K)
    dst5 = edge_index[1].astype(jnp.int32).reshape(NS, NPH, PH, K)
    bn_gs = bn_g / jnp.sqrt(1.0 + 1e-5)

    h0, h1 = _in_proj(x, in_W, in_b)
    ea0, ea1 = _edge_proj(edge_attr, ep_W)
    for i in range(DEPTH):
        a0, a1 = _edge_phase(h0, h1, ea0, ea1, src5, dst5)
        h0, h1 = _mlp(h0, h1, a0, a1, W1[i], b1[i], W2[i], b2[i],
                      ln_g[i], ln_b[i], bn_gs[i], bn_b[i])
    return _pool_readout(batch.astype(jnp.int32), h0, h1, ro_W, ro_b)


# speedup vs baseline: 5.0423x; 1.3866x over previous
import functools

import jax
import jax.numpy as jnp
from jax import lax
from jax.experimental import pallas as pl
from jax.experimental.pallas import tpu as pltpu
from jax.experimental.pallas import tpu_sc as plsc

N = 10000
E = 320000
NODE_IN = 128
EDGE_IN = 16
H = 256
HH = 128
G = 64
DEPTH = 5

NC, NS, L = 2, 16, 16
K = 40
NBUF = 5
PH = 50
EPS = E // NS
NPH = EPS // (K * PH)
ZR = 200


def _in_proj_body(x_ref, w_ref, b_ref, o0, o1):
    z = jnp.dot(x_ref[...], w_ref[...], preferred_element_type=jnp.float32)
    z = z + b_ref[...]
    o0[...] = z[:, :HH]
    o1[...] = z[:, HH:]


def _in_proj(x, in_W, in_b):
    bn = 2000
    return pl.pallas_call(
        _in_proj_body,
        grid=(N // bn,),
        in_specs=[
            pl.BlockSpec((bn, NODE_IN), lambda i: (i, 0)),
            pl.BlockSpec((NODE_IN, H), lambda i: (0, 0)),
            pl.BlockSpec((1, H), lambda i: (0, 0)),
        ],
        out_specs=[pl.BlockSpec((bn, HH), lambda i: (i, 0))] * 2,
        out_shape=[jax.ShapeDtypeStruct((N, HH), jnp.float32)] * 2,
    )(x, in_W, in_b.reshape(1, H))


def _edge_proj_body(a_ref, w_ref, o0, o1):
    z = jnp.dot(a_ref[...], w_ref[...], preferred_element_type=jnp.float32)
    o0[...] = z[:, :HH]
    o1[...] = z[:, HH:]


def _edge_proj(edge_attr, ep_W):
    be = 4000
    return pl.pallas_call(
        _edge_proj_body,
        grid=(E // be,),
        in_specs=[
            pl.BlockSpec((be, EDGE_IN), lambda i: (i, 0)),
            pl.BlockSpec((EDGE_IN, H), lambda i: (0, 0)),
        ],
        out_specs=[pl.BlockSpec((be, HH), lambda i: (i, 0))] * 2,
        out_shape=[jax.ShapeDtypeStruct((E, HH), jnp.float32)] * 2,
    )(edge_attr, ep_W)


def _mlp_body(h0, h1, a0, a1, w1, b1, w2, b2, lng, lnb, bng, bnb, o0, o1):
    h = jnp.concatenate([h0[...], h1[...]], axis=1)
    z = h + jnp.concatenate([a0[...], a1[...]], axis=1)
    z = jnp.dot(z, w1[...], preferred_element_type=jnp.float32) + b1[...]
    z = jnp.maximum(z, 0.0)
    z = jnp.dot(z, w2[...], preferred_element_type=jnp.float32) + b2[...]
    m = jnp.mean(z, axis=1, keepdims=True)
    v = jnp.mean((z - m) ** 2, axis=1, keepdims=True)
    z = (z - m) * lax.rsqrt(v + 1e-5) * lng[...] + lnb[...]
    z = z * bng[...] + bnb[...]
    hn = h + jnp.maximum(z, 0.0)
    o0[...] = hn[:, :HH]
    o1[...] = hn[:, HH:]


def _mlp(h0, h1, a0, a1, w1, b1, w2, b2, lng, lnb, bng, bnb):
    bn = 2000
    vec = lambda: pl.BlockSpec((1, H), lambda i: (0, 0))
    half = lambda: pl.BlockSpec((bn, HH), lambda i: (i, 0))
    return pl.pallas_call(
        _mlp_body,
        grid=(N // bn,),
        in_specs=[half(), half(), half(), half(),
                  pl.BlockSpec((H, H), lambda i: (0, 0)), vec(),
                  pl.BlockSpec((H, H), lambda i: (0, 0)), vec(),
                  vec(), vec(), vec(), vec()],
        out_specs=[half()] * 2,
        out_shape=[jax.ShapeDtypeStruct((N, HH), jnp.float32)] * 2,
    )(h0, h1, a0, a1, w1, b1.reshape(1, H), w2, b2.reshape(1, H),
      lng.reshape(1, H), lnb.reshape(1, H), bng.reshape(1, H),
      bnb.reshape(1, H))


def _pool_body(b_ref, h0, h1, row, rob, o_ref, sums, cnts):
    i = pl.program_id(0)

    @pl.when(i == 0)
    def _():
        sums[...] = jnp.zeros_like(sums)
        cnts[...] = jnp.zeros_like(cnts)

    bb = b_ref[0, 0, :]
    bn = bb.shape[0]
    oh = (bb[None, :] == lax.broadcasted_iota(jnp.int32, (G, bn), 0))
    oh = oh.astype(jnp.float32)
    h = jnp.concatenate([h0[...], h1[...]], axis=1)
    sums[...] += jnp.dot(oh, h, preferred_element_type=jnp.float32)
    cnts[...] += jnp.sum(oh, axis=1, keepdims=True)

    @pl.when(i == pl.num_programs(0) - 1)
    def _():
        g = sums[...] / jnp.maximum(cnts[...][:, 0:1], 1.0)
        o_ref[...] = (jnp.dot(g, row[...], preferred_element_type=jnp.float32)
                      + rob[...])


def _pool_readout(batch, h0, h1, ro_W, ro_b):
    bn = 1000
    nb = N // bn
    half = lambda: pl.BlockSpec((bn, HH), lambda i: (i, 0))
    return pl.pallas_call(
        _pool_body,
        grid=(nb,),
        in_specs=[
            pl.BlockSpec((1, 1, bn), lambda i: (i, 0, 0)),
            half(), half(),
            pl.BlockSpec((H, H), lambda i: (0, 0)),
            pl.BlockSpec((1, H), lambda i: (0, 0)),
        ],
        out_specs=pl.BlockSpec((G, H), lambda i: (0, 0)),
        out_shape=jax.ShapeDtypeStruct((G, H), jnp.float32),
        scratch_shapes=[
            pltpu.VMEM((G, H), jnp.float32),
            pltpu.VMEM((G, 128), jnp.float32),
        ],
    )(batch.reshape(nb, 1, bn), h0, h1, ro_W, ro_b.reshape(1, H))


def _edge_sc_body(h0, h1, ea0, ea1, src5, dst5, o0, o1,
                  sidx, didx, b0, b1, b2, b3, b4,
                  se0, se1, se2, se3, se4,
                  sg0, sg1, sg2, sg3, sg4,
                  ss0, ss1, ss2, ss3, ss4, acc):
    c = lax.axis_index("c")
    s = lax.axis_index("s")
    bufs = (b0, b1, b2, b3, b4)
    semE = (se0, se1, se2, se3, se4)
    semG = (sg0, sg1, sg2, sg3, sg4)
    semS = (ss0, ss1, ss2, ss3, ss4)

    def run(h_hbm, ea_hbm, out_hbm):
        def zrow(r, _):
            for cc in range(HH // L):
                b0[r, pl.ds(cc * L, L)] = jnp.zeros((L,), jnp.float32)
            return 0

        lax.fori_loop(0, K, zrow, 0, unroll=2)
        for t in range((N // K + NS - 1) // NS):
            ch = s + t * NS

            @pl.when(ch < N // K)
            def _():
                pltpu.sync_copy(b0, acc.at[pl.ds(ch * K, K)])

        plsc.subcore_barrier()

        def ea_start(g, b):
            pltpu.async_copy(ea_hbm.at[pl.ds(s * EPS + g * K, K)],
                             bufs[b], semE[b])

        def ea_wait(b):
            pltpu.make_async_copy(ea_hbm.at[pl.ds(0, K)], bufs[b],
                                  semE[b]).wait()

        def g_start(jj, b):
            pltpu.async_copy(h_hbm.at[sidx.at[jj]], bufs[b], semG[b],
                             add=True)

        def g_wait(b):
            pltpu.make_async_copy(h_hbm.at[pl.ds(0, K)], bufs[b],
                                  semG[b]).wait()

        def s_start(jj, b):
            pltpu.async_copy(bufs[b], acc.at[didx.at[jj]], semS[b],
                             add=True)

        def s_wait(b):
            pltpu.make_async_copy(bufs[b], acc.at[pl.ds(0, K)],
                                  semS[b]).wait()

        def relu(b):
            buf = bufs[b]

            def row(r, _):
                for cc in range(HH // L):
                    v = buf[r, pl.ds(cc * L, L)]
                    buf[r, pl.ds(cc * L, L)] = jnp.maximum(v, 0.0)
                return 0

            lax.fori_loop(0, K, row, 0, unroll=2)

        def phase(p, _):
            jb = p * PH
            pltpu.sync_copy(src5.at[s, p], sidx)
            pltpu.sync_copy(dst5.at[s, p], didx)
            ea_start(jb + 0, 0)
            ea_start(jb + 1, 1)

            def iter_t(t, _):
                for b in range(NBUF):
                    jj = t * NBUF + b
                    bm2 = (b - 2) % NBUF
                    bm3 = (b - 3) % NBUF
                    bp2 = (b + 2) % NBUF
                    ea_wait(b)
                    g_start(jj, b)

                    def do_b():
                        g_wait(bm2)
                        relu(bm2)
                        s_start(jj - 2, bm2)

                    if b >= 2:
                        do_b()
                    else:
                        @pl.when(t > 0)
                        def _():
                            do_b()

                    def do_cw():
                        s_wait(bm3)

                    if b >= 3:
                        do_cw()
                    else:
                        @pl.when(t > 0)
                        def _():
                            do_cw()

                    if b <= 2:
                        ea_start(jb + jj + 2, bp2)
                    else:
                        @pl.when(t < PH // NBUF - 1)
                        def _():
                            ea_start(jb + jj + 2, bp2)
                return 0

            lax.fori_loop(0, PH // NBUF, iter_t, 0)
            for jj, b in ((PH - 2, (PH - 2) % NBUF), (PH - 1, (PH - 1) % NBUF)):
                g_wait(b)
                relu(b)
                s_start(jj, b)
            for jj in (PH - 3, PH - 2, PH - 1):
                s_wait(jj % NBUF)
            return 0

        lax.fori_loop(0, NPH, phase, 0)
        plsc.subcore_barrier()
        for t in range((N // ZR + NS - 1) // NS):
            ch = s + t * NS

            @pl.when(ch < N // ZR)
            def _():
                pltpu.sync_copy(acc.at[pl.ds(ch * ZR, ZR)],
                                out_hbm.at[pl.ds(ch * ZR, ZR)])

    @pl.when(c == 0)
    def _():
        run(h0, ea0, o0)

    @pl.when(c == 1)
    def _():
        run(h1, ea1, o1)


@functools.partial(
    pl.kernel,
    out_type=(jax.ShapeDtypeStruct((N, HH), jnp.float32),
              jax.ShapeDtypeStruct((N, HH), jnp.float32)),
    mesh=plsc.VectorSubcoreMesh(core_axis_name="c", subcore_axis_name="s"),
    scratch_types=(
        [pltpu.VMEM((PH, K), jnp.int32)] * 2
        + [pltpu.VMEM((K, HH), jnp.float32)] * NBUF
        + [pltpu.SemaphoreType.DMA] * (3 * NBUF)
        + [pltpu.VMEM_SHARED((N, HH), jnp.float32)]
    ),
)
def _edge_phase(h0, h1, ea0, ea1, src5, dst5, o0, o1,
                sidx, didx, b0, b1, b2, b3, b4,
                se0, se1, se2, se3, se4,
                sg0, sg1, sg2, sg3, sg4,
                ss0, ss1, ss2, ss3, ss4, acc):
    _edge_sc_body(h0, h1, ea0, ea1, src5, dst5, o0, o1,
                  sidx, didx, b0, b1, b2, b3, b4,
                  se0, se1, se2, se3, se4,
                  sg0, sg1, sg2, sg3, sg4,
                  ss0, ss1, ss2, ss3, ss4, acc)


@jax.jit
def kernel(x, edge_index, edge_attr, batch, in_W, in_b, ep_W, W1, b1,
           W2, b2, ln_g, ln_b, bn_g, bn_b, ro_W, ro_b):
    src5 = edge_index[0].astype(jnp.int32).reshape(NS, NPH, PH, K)
    dst5 = edge_index[1].astype(jnp.int32).reshape(NS, NPH, PH, K)
    bn_gs = bn_g / jnp.sqrt(1.0 + 1e-5)

    h0, h1 = _in_proj(x, in_W, in_b)
    ea0, ea1 = _edge_proj(edge_attr, ep_W)
    for i in range(DEPTH):
        a0, a1 = _edge_phase(h0, h1, ea0, ea1, src5, dst5)
        h0, h1 = _mlp(h0, h1, a0, a1, W1[i], b1[i], W2[i], b2[i],
                      ln_g[i], ln_b[i], bn_gs[i], bn_b[i])
    return _pool_readout(batch.astype(jnp.int32), h0, h1, ro_W, ro_b)
